# baseline (device time: 43811 ns/iter reference)
import jax
import jax.numpy as jnp
from jax import lax
from jax.experimental import pallas as pl
from jax.experimental.pallas import tpu as pltpu

N_DEV = 4
SQ = 1024
D = 1024
H = 8
DH = 128
BLK = 64
SCALE = 0.08838834764831843

NC = 4
CHK = SQ // NC

SEND0 = {
    0: [(1, 0)],
    1: [(3, 1)],
    2: [(1, 0), (3, 1)],
    3: [(3, 1), (1, 0)],
}
SCHED = {
    1: [(0, 2), (2, 2), (3, None), (1, None)],
    2: [(0, 3), (1, 1), (2, None), (3, None)],
    3: [(1, 2), (3, 2), (2, None), (0, None)],
}


def kernel(x, Wq, K_ext, V_ext, Wo):
    skv = K_ext.shape[1]
    kf = K_ext.reshape(skv, D)
    vf = V_ext.reshape(skv, D)

    def body(x_ref, wq_ref, k_ref, v_ref, wo_ref, out_ref,
             ctx_ref, send_sems, recv_sems):
        my = lax.axis_index("i")

        barrier = pltpu.get_barrier_semaphore()
        for d in range(N_DEV):
            @pl.when(my != d)
            def _(d=d):
                pl.semaphore_signal(
                    barrier, inc=1,
                    device_id=(d,), device_id_type=pl.DeviceIdType.MESH,
                )
        pl.semaphore_wait(barrier, N_DEV - 1)

        def chunk_rdma(c, target, slot=0):
            return pltpu.make_async_remote_copy(
                src_ref=ctx_ref.at[c], dst_ref=ctx_ref.at[c],
                send_sem=send_sems.at[c, slot], recv_sem=recv_sems.at[c],
                device_id=(target,), device_id_type=pl.DeviceIdType.MESH,
            )

        def wo_chunk(c, wob):
            out_ref[0, c * CHK:(c + 1) * CHK, :] = jnp.dot(
                ctx_ref[c], wob, preferred_element_type=jnp.float32)

        @pl.when(my == 0)
        def _():
            kkb = k_ref[...].astype(jnp.bfloat16)
            vvb = v_ref[...].astype(jnp.bfloat16)
            xb = x_ref[0].astype(jnp.bfloat16)
            wqb = wq_ref[...].astype(jnp.bfloat16)

            def qproj(lo, hi):
                return (jnp.dot(xb[lo:hi, :], wqb,
                                preferred_element_type=jnp.float32)
                        * SCALE).astype(jnp.bfloat16)

            q = qproj(0, SQ // 2)
            for c in range(NC):
                if c == NC // 2:
                    q = qproj(SQ // 2, SQ)
                nk = CHK * (c + 1)
                qc = q[(c % (NC // 2)) * CHK:(c % (NC // 2) + 1) * CHK, :]
                rb = (lax.broadcasted_iota(jnp.int32, (CHK, nk), 0)
                      + c * CHK) // BLK
                cb = lax.broadcasted_iota(jnp.int32, (CHK, nk), 1) // BLK
                msk = cb <= rb
                for h in range(H):
                    hs = slice(h * DH, (h + 1) * DH)
                    s = lax.dot_general(
                        qc[:, hs], kkb[:nk, hs],
                        (((1,), (1,)), ((), ())),
                        preferred_element_type=jnp.float32,
                    )
                    w = jnp.where(msk, jnp.exp(s), 0.0)
                    l = jnp.sum(w, axis=1, keepdims=True)
                    ctx_ref[c, :, hs] = (lax.dot_general(
                        w.astype(jnp.bfloat16), vvb[:nk, hs],
                        (((1,), (0,)), ((), ())),
                        preferred_element_type=jnp.float32,
                    ) / l).astype(jnp.bfloat16)
                for tgt, slot in SEND0[c]:
                    chunk_rdma(c, tgt, slot).start()
            wob = wo_ref[...].astype(jnp.bfloat16)
            for c in range(NC):
                wo_chunk(c, wob)
            for c, routes in SEND0.items():
                for _, slot in routes:
                    chunk_rdma(c, 1, slot).wait_send()

        for dev, sched in SCHED.items():
            @pl.when(my == dev)
            def _(sched=sched):
                wob = wo_ref[...].astype(jnp.bfloat16)
                for c, fwd in sched:
                    chunk_rdma(c, 0).wait_recv()
                    if fwd is not None:
                        chunk_rdma(c, fwd).start()
                    wo_chunk(c, wob)
                for c, fwd in sched:
                    if fwd is not None:
                        chunk_rdma(c, fwd).wait_send()

    return pl.pallas_call(
        body,
        out_shape=jax.ShapeDtypeStruct((1, SQ, D), jnp.float32),
        in_specs=[pl.BlockSpec(memory_space=pltpu.VMEM)] * 5,
        out_specs=pl.BlockSpec(memory_space=pltpu.VMEM),
        scratch_shapes=[
            pltpu.VMEM((NC, CHK, D), jnp.bfloat16),
            pltpu.SemaphoreType.DMA((NC, 2)),
            pltpu.SemaphoreType.DMA((NC,)),
        ],
        compiler_params=pltpu.CompilerParams(collective_id=0),
    )(x, Wq, kf, vf, Wo)


# device time: 40032 ns/iter; 1.0944x vs baseline; 1.0944x over previous
import jax
import jax.numpy as jnp
from jax import lax
from jax.experimental import pallas as pl
from jax.experimental.pallas import tpu as pltpu

N_DEV = 4
SQ = 1024
D = 1024
H = 8
DH = 128
BLK = 64
SCALE = 0.08838834764831843

NC = 8
CHK = SQ // NC

SEND0 = {
    0: [(1, 0)], 2: [(1, 0)],
    1: [(3, 1)], 3: [(3, 1)],
    4: [(1, 0), (3, 1)], 6: [(1, 0), (3, 1)],
    5: [(3, 1), (1, 0)], 7: [(3, 1), (1, 0)],
}
SCHED = {
    1: [(0, 2), (2, 2), (4, 2), (6, 2), (5, None), (7, None),
        (1, None), (3, None)],
    2: [(0, 3), (1, 1), (2, 3), (3, 1), (4, None), (5, None),
        (6, None), (7, None)],
    3: [(1, 2), (3, 2), (5, 2), (7, 2), (4, None), (6, None),
        (0, None), (2, None)],
}


def kernel(x, Wq, K_ext, V_ext, Wo):
    skv = K_ext.shape[1]
    kf = K_ext.reshape(skv, D)
    vf = V_ext.reshape(skv, D)

    def body(x_ref, wq_ref, k_ref, v_ref, wo_ref, out_ref,
             ctx_ref, send_sems, recv_sems):
        my = lax.axis_index("i")

        barrier = pltpu.get_barrier_semaphore()
        for d in range(N_DEV):
            @pl.when(my != d)
            def _(d=d):
                pl.semaphore_signal(
                    barrier, inc=1,
                    device_id=(d,), device_id_type=pl.DeviceIdType.MESH,
                )
        pl.semaphore_wait(barrier, N_DEV - 1)

        def chunk_rdma(c, target, slot=0):
            return pltpu.make_async_remote_copy(
                src_ref=ctx_ref.at[c], dst_ref=ctx_ref.at[c],
                send_sem=send_sems.at[c, slot], recv_sem=recv_sems.at[c],
                device_id=(target,), device_id_type=pl.DeviceIdType.MESH,
            )

        def wo_chunk(c, wob):
            out_ref[0, c * CHK:(c + 1) * CHK, :] = jnp.dot(
                ctx_ref[c], wob, preferred_element_type=jnp.float32)

        @pl.when(my == 0)
        def _():
            kkb = k_ref[...].astype(jnp.bfloat16)
            vvb = v_ref[...].astype(jnp.bfloat16)
            xb = x_ref[0].astype(jnp.bfloat16)
            wqb = wq_ref[...].astype(jnp.bfloat16)

            def qproj(lo, hi):
                return (jnp.dot(xb[lo:hi, :], wqb,
                                preferred_element_type=jnp.float32)
                        * SCALE).astype(jnp.bfloat16)

            q = qproj(0, SQ // 2)
            for c in range(NC):
                if c == NC // 2:
                    q = qproj(SQ // 2, SQ)
                nk = CHK * (c + 1)
                qc = q[(c % (NC // 2)) * CHK:(c % (NC // 2) + 1) * CHK, :]
                rb = (lax.broadcasted_iota(jnp.int32, (CHK, nk), 0)
                      + c * CHK) // BLK
                cb = lax.broadcasted_iota(jnp.int32, (CHK, nk), 1) // BLK
                msk = cb <= rb
                for h in range(H):
                    hs = slice(h * DH, (h + 1) * DH)
                    s = lax.dot_general(
                        qc[:, hs], kkb[:nk, hs],
                        (((1,), (1,)), ((), ())),
                        preferred_element_type=jnp.float32,
                    )
                    w = jnp.where(msk, jnp.exp(s), 0.0)
                    l = jnp.sum(w, axis=1, keepdims=True)
                    ctx_ref[c, :, hs] = (lax.dot_general(
                        w.astype(jnp.bfloat16), vvb[:nk, hs],
                        (((1,), (0,)), ((), ())),
                        preferred_element_type=jnp.float32,
                    ) / l).astype(jnp.bfloat16)
                for tgt, slot in SEND0[c]:
                    chunk_rdma(c, tgt, slot).start()
            wob = wo_ref[...].astype(jnp.bfloat16)
            for c in range(NC):
                wo_chunk(c, wob)
            for c, routes in SEND0.items():
                for _, slot in routes:
                    chunk_rdma(c, 1, slot).wait_send()

        for dev, sched in SCHED.items():
            @pl.when(my == dev)
            def _(sched=sched):
                wob = wo_ref[...].astype(jnp.bfloat16)
                for c, fwd in sched:
                    chunk_rdma(c, 0).wait_recv()
                    if fwd is not None:
                        chunk_rdma(c, fwd).start()
                    wo_chunk(c, wob)
                for c, fwd in sched:
                    if fwd is not None:
                        chunk_rdma(c, fwd).wait_send()

    return pl.pallas_call(
        body,
        out_shape=jax.ShapeDtypeStruct((1, SQ, D), jnp.float32),
        in_specs=[pl.BlockSpec(memory_space=pltpu.VMEM)] * 5,
        out_specs=pl.BlockSpec(memory_space=pltpu.VMEM),
        scratch_shapes=[
            pltpu.VMEM((NC, CHK, D), jnp.bfloat16),
            pltpu.SemaphoreType.DMA((NC, 2)),
            pltpu.SemaphoreType.DMA((NC,)),
        ],
        compiler_params=pltpu.CompilerParams(collective_id=0),
    )(x, Wq, kf, vf, Wo)
